# baseline (speedup 1.0000x reference)
import functools

import jax
import jax.numpy as jnp
from jax import lax
from jax.experimental import pallas as pl
from jax.experimental.pallas import tpu as pltpu
from jax.experimental.pallas import tpu_sc as plsc

N_NODES = 10000
D = 128
EPS = 1e-5

NC = 2
NS = 16
NW = NC * NS
C = 64
NCHUNK = 160
DEPTH = 4
EPT = C * NCHUNK
E_PAD = NW * EPT
AGG_ROWS = 10240
ZROWS_PER_TILE = AGG_ROWS // NS


def _matT(x, w):
    return lax.dot_general(x, w, (((1,), (1,)), ((), ())),
                           preferred_element_type=jnp.float32)


def _lin_body(x_ref, w_ref, b_ref, o_ref):
    o_ref[...] = _matT(x_ref[...], w_ref[...]) + b_ref[...][None, :]


def _tc_linear(x, w, b):
    nblk = 10
    rows = N_NODES // nblk
    return pl.pallas_call(
        _lin_body,
        grid=(nblk,),
        in_specs=[
            pl.BlockSpec((rows, D), lambda i: (i, 0)),
            pl.BlockSpec((D, D), lambda i: (0, 0)),
            pl.BlockSpec((D,), lambda i: (0,)),
        ],
        out_specs=pl.BlockSpec((rows, D), lambda i: (i, 0)),
        out_shape=jax.ShapeDtypeStruct((N_NODES, D), jnp.float32),
    )(x, w, b)


@functools.lru_cache(maxsize=1)
def _sc_scatter_build():
    mesh = plsc.VectorSubcoreMesh(core_axis_name="c", subcore_axis_name="s")

    scratch = (
        [pltpu.VMEM((NCHUNK // 4, C), jnp.int32)] * 2
        + [pltpu.VMEM((C, D), jnp.float32)] * DEPTH
        + [pltpu.VMEM_SHARED((AGG_ROWS, D), jnp.float32)]
        + [pltpu.SemaphoreType.DMA] * (2 * DEPTH)
    )

    @functools.partial(
        pl.kernel,
        mesh=mesh,
        out_type=jax.ShapeDtypeStruct((NC, AGG_ROWS, D), jnp.float32),
        scratch_types=scratch,
    )
    def sc_scatter(h_hbm, src_hbm, dst_hbm, zrows_hbm, out_hbm,
                   sidx, didx, *rest):
        bufs = rest[:DEPTH]
        agg = rest[DEPTH]
        gsem = rest[DEPTH + 1:DEPTH + 1 + DEPTH]
        ssem = rest[DEPTH + 1 + DEPTH:]
        cid = lax.axis_index("c")
        sid = lax.axis_index("s")
        wid = cid * NS + sid

        pltpu.sync_copy(zrows_hbm,
                        agg.at[pl.ds(sid * ZROWS_PER_TILE, ZROWS_PER_TILE)])
        plsc.subcore_barrier()

        def gat(k, b):
            return pltpu.make_async_copy(h_hbm.at[sidx.at[k]], bufs[b], gsem[b])

        def sca(k, b):
            return pltpu.make_async_copy(bufs[b], agg.at[didx.at[k]], ssem[b])

        half = NCHUNK // 4
        nit = half // DEPTH
        for hh in range(4):
            pltpu.sync_copy(src_hbm.at[wid, pl.ds(hh * half, half)], sidx)
            pltpu.sync_copy(dst_hbm.at[wid, pl.ds(hh * half, half)], didx)
            for b in range(DEPTH):
                gat(b, b).start()

            def body(j, carry):
                for b in range(DEPTH):
                    k = DEPTH * j + b
                    gat(k, b).wait()
                    sca(k, b).start(add=True)
                for b in range(DEPTH):
                    k = DEPTH * j + b
                    sca(k, b).wait()

                    @pl.when(j < nit - 1)
                    def _prefetch():
                        gat(k + DEPTH, b).start()

                return carry

            lax.fori_loop(0, nit, body, 0)
        plsc.subcore_barrier()

        r0 = sid * ZROWS_PER_TILE
        pltpu.sync_copy(agg.at[pl.ds(r0, ZROWS_PER_TILE)],
                        out_hbm.at[cid, pl.ds(r0, ZROWS_PER_TILE)])

    return sc_scatter


_NBLK = 10
_ROWS = N_NODES // _NBLK


def _comb_body(p_ref, x_ref, w_ref, b_ref, g_ref, bt_ref, o_ref,
               t_ref, s_ref, q_ref):
    ph = pl.program_id(0)
    i = pl.program_id(1)

    @pl.when(ph == 0)
    def _compute():
        new = jnp.maximum(p_ref[0] + p_ref[1], 0.0)
        res = jnp.maximum(_matT(x_ref[...], w_ref[...]) + b_ref[...][None, :],
                          0.0)
        t = new + res
        t_ref[pl.ds(i * _ROWS, _ROWS), :] = t
        s_ref[pl.ds(i, 1), :] = jnp.sum(t, axis=0, keepdims=True)
        q_ref[pl.ds(i, 1), :] = jnp.sum(t * t, axis=0, keepdims=True)

    @pl.when(ph == 1)
    def _normalize():
        n = float(N_NODES)
        mean = jnp.sum(s_ref[...], axis=0, keepdims=True) / n
        var = jnp.sum(q_ref[...], axis=0, keepdims=True) / n - mean * mean
        inv = lax.rsqrt(var + EPS)
        t = t_ref[pl.ds(i * _ROWS, _ROWS), :]
        o_ref[...] = (t - mean) * (inv * g_ref[...][None, :]) + bt_ref[...][None, :]


def _tc_combine(p, x, w, b, gamma, beta):
    return pl.pallas_call(
        _comb_body,
        grid=(2, _NBLK),
        in_specs=[
            pl.BlockSpec((NC, _ROWS, D), lambda p_, i: (0, (1 - p_) * i, 0)),
            pl.BlockSpec((_ROWS, D), lambda p_, i: ((1 - p_) * i, 0)),
            pl.BlockSpec((D, D), lambda p_, i: (0, 0)),
            pl.BlockSpec((D,), lambda p_, i: (0,)),
            pl.BlockSpec((D,), lambda p_, i: (0,)),
            pl.BlockSpec((D,), lambda p_, i: (0,)),
        ],
        out_specs=pl.BlockSpec((_ROWS, D), lambda p_, i: (p_ * i, 0)),
        out_shape=jax.ShapeDtypeStruct((N_NODES, D), jnp.float32),
        scratch_shapes=[
            pltpu.VMEM((N_NODES, D), jnp.float32),
            pltpu.VMEM((_NBLK, D), jnp.float32),
            pltpu.VMEM((_NBLK, D), jnp.float32),
        ],
    )(p, x, w, b, gamma, beta)


def kernel(feats, edge_index, W_rel, b_rel, W_res, b_res, gamma, beta):
    src = edge_index[0].astype(jnp.int32)
    dst = edge_index[1].astype(jnp.int32)
    pad = E_PAD - src.shape[0]
    pad_iota = jnp.arange(pad, dtype=jnp.int32)
    src = jnp.concatenate([src, pad_iota % N_NODES])
    dst = jnp.concatenate([dst, N_NODES + pad_iota % (AGG_ROWS - N_NODES)])
    src3 = src.reshape(NW, NCHUNK, C)
    dst3 = dst.reshape(NW, NCHUNK, C)
    zrows = jnp.zeros((ZROWS_PER_TILE, D), jnp.float32)

    h = _tc_linear(feats, W_rel, b_rel)
    p = _sc_scatter_build()(h, src3, dst3, zrows)
    return _tc_combine(p, feats, W_res, b_res, gamma, beta)


# speedup vs baseline: 1.0295x; 1.0295x over previous
import functools

import jax
import jax.numpy as jnp
from jax import lax
from jax.experimental import pallas as pl
from jax.experimental.pallas import tpu as pltpu
from jax.experimental.pallas import tpu_sc as plsc

N_NODES = 10000
D = 128
EPS = 1e-5

NC = 2
NS = 16
NW = NC * NS
C = 64
NCHUNK = 160
DEPTH = 4
EPT = C * NCHUNK
E_PAD = NW * EPT
AGG_ROWS = 10240
ZROWS_PER_TILE = AGG_ROWS // NS


def _matT(x, w):
    return lax.dot_general(x, w, (((1,), (1,)), ((), ())),
                           preferred_element_type=jnp.float32)


def _lin_body(x_ref, w_ref, b_ref, o_ref):
    o_ref[...] = _matT(x_ref[...], w_ref[...]) + b_ref[...][None, :]


def _tc_linear(x, w, b):
    nblk = 10
    rows = N_NODES // nblk
    return pl.pallas_call(
        _lin_body,
        grid=(nblk,),
        in_specs=[
            pl.BlockSpec((rows, D), lambda i: (i, 0)),
            pl.BlockSpec((D, D), lambda i: (0, 0)),
            pl.BlockSpec((D,), lambda i: (0,)),
        ],
        out_specs=pl.BlockSpec((rows, D), lambda i: (i, 0)),
        out_shape=jax.ShapeDtypeStruct((N_NODES, D), jnp.float32),
    )(x, w, b)


@functools.lru_cache(maxsize=1)
def _sc_scatter_build():
    mesh = plsc.VectorSubcoreMesh(core_axis_name="c", subcore_axis_name="s")

    scratch = (
        [pltpu.VMEM((NCHUNK // 4, C), jnp.int32)] * 2
        + [pltpu.VMEM((C, D), jnp.float32)] * DEPTH
        + [pltpu.VMEM_SHARED((AGG_ROWS, D), jnp.float32)]
        + [pltpu.SemaphoreType.DMA] * (2 * DEPTH)
    )

    @functools.partial(
        pl.kernel,
        mesh=mesh,
        out_type=jax.ShapeDtypeStruct((NC, AGG_ROWS, D), jnp.float32),
        scratch_types=scratch,
    )
    def sc_scatter(h_hbm, src_hbm, dst_hbm, zrows_hbm, out_hbm,
                   sidx, didx, *rest):
        bufs = rest[:DEPTH]
        agg = rest[DEPTH]
        gsem = rest[DEPTH + 1:DEPTH + 1 + DEPTH]
        ssem = rest[DEPTH + 1 + DEPTH:]
        cid = lax.axis_index("c")
        sid = lax.axis_index("s")
        wid = cid * NS + sid

        pltpu.sync_copy(zrows_hbm, bufs[0])
        for k in range(ZROWS_PER_TILE // C):
            pltpu.sync_copy(bufs[0],
                            agg.at[pl.ds(sid * ZROWS_PER_TILE + k * C, C)])
        plsc.subcore_barrier()

        def gat(k, b):
            return pltpu.make_async_copy(h_hbm.at[sidx.at[k]], bufs[b], gsem[b])

        def sca(k, b):
            return pltpu.make_async_copy(bufs[b], agg.at[didx.at[k]], ssem[b])

        half = NCHUNK // 4
        nit = half // DEPTH
        for hh in range(4):
            pltpu.sync_copy(src_hbm.at[wid, pl.ds(hh * half, half)], sidx)
            pltpu.sync_copy(dst_hbm.at[wid, pl.ds(hh * half, half)], didx)
            for b in range(DEPTH):
                gat(b, b).start()

            def body(j, carry):
                for b in range(DEPTH):
                    k = DEPTH * j + b
                    gat(k, b).wait()
                    sca(k, b).start(add=True)
                for b in range(DEPTH):
                    k = DEPTH * j + b
                    sca(k, b).wait()

                    @pl.when(j < nit - 1)
                    def _prefetch():
                        gat(k + DEPTH, b).start()

                return carry

            lax.fori_loop(0, nit, body, 0)
        plsc.subcore_barrier()

        r0 = sid * ZROWS_PER_TILE
        pltpu.sync_copy(agg.at[pl.ds(r0, ZROWS_PER_TILE)],
                        out_hbm.at[cid, pl.ds(r0, ZROWS_PER_TILE)])

    return sc_scatter


_NBLK = 10
_ROWS = N_NODES // _NBLK


def _comb_body(p_ref, x_ref, w_ref, b_ref, g_ref, bt_ref, o_ref,
               t_ref, s_ref, q_ref):
    ph = pl.program_id(0)
    i = pl.program_id(1)

    @pl.when(ph == 0)
    def _compute():
        new = jnp.maximum(p_ref[0] + p_ref[1], 0.0)
        res = jnp.maximum(_matT(x_ref[...], w_ref[...]) + b_ref[...][None, :],
                          0.0)
        t = new + res
        t_ref[pl.ds(i * _ROWS, _ROWS), :] = t
        s_ref[pl.ds(i, 1), :] = jnp.sum(t, axis=0, keepdims=True)
        q_ref[pl.ds(i, 1), :] = jnp.sum(t * t, axis=0, keepdims=True)

    @pl.when(ph == 1)
    def _normalize():
        n = float(N_NODES)
        mean = jnp.sum(s_ref[...], axis=0, keepdims=True) / n
        var = jnp.sum(q_ref[...], axis=0, keepdims=True) / n - mean * mean
        inv = lax.rsqrt(var + EPS)
        t = t_ref[pl.ds(i * _ROWS, _ROWS), :]
        o_ref[...] = (t - mean) * (inv * g_ref[...][None, :]) + bt_ref[...][None, :]


def _tc_combine(p, x, w, b, gamma, beta):
    return pl.pallas_call(
        _comb_body,
        grid=(2, _NBLK),
        in_specs=[
            pl.BlockSpec((NC, _ROWS, D), lambda p_, i: (0, (1 - p_) * i, 0)),
            pl.BlockSpec((_ROWS, D), lambda p_, i: ((1 - p_) * i, 0)),
            pl.BlockSpec((D, D), lambda p_, i: (0, 0)),
            pl.BlockSpec((D,), lambda p_, i: (0,)),
            pl.BlockSpec((D,), lambda p_, i: (0,)),
            pl.BlockSpec((D,), lambda p_, i: (0,)),
        ],
        out_specs=pl.BlockSpec((_ROWS, D), lambda p_, i: (p_ * i, 0)),
        out_shape=jax.ShapeDtypeStruct((N_NODES, D), jnp.float32),
        scratch_shapes=[
            pltpu.VMEM((N_NODES, D), jnp.float32),
            pltpu.VMEM((_NBLK, D), jnp.float32),
            pltpu.VMEM((_NBLK, D), jnp.float32),
        ],
    )(p, x, w, b, gamma, beta)


def kernel(feats, edge_index, W_rel, b_rel, W_res, b_res, gamma, beta):
    src = edge_index[0].astype(jnp.int32)
    dst = edge_index[1].astype(jnp.int32)
    pad = E_PAD - src.shape[0]
    pad_iota = jnp.arange(pad, dtype=jnp.int32)
    src = jnp.concatenate([src, pad_iota % N_NODES])
    dst = jnp.concatenate([dst, N_NODES + pad_iota % (AGG_ROWS - N_NODES)])
    src3 = src.reshape(NW, NCHUNK, C)
    dst3 = dst.reshape(NW, NCHUNK, C)
    zrows = jnp.zeros((C, D), jnp.float32)

    h = _tc_linear(feats, W_rel, b_rel)
    p = _sc_scatter_build()(h, src3, dst3, zrows)
    return _tc_combine(p, feats, W_res, b_res, gamma, beta)


# speedup vs baseline: 1.0505x; 1.0204x over previous
import functools

import jax
import jax.numpy as jnp
from jax import lax
from jax.experimental import pallas as pl
from jax.experimental.pallas import tpu as pltpu
from jax.experimental.pallas import tpu_sc as plsc

N_NODES = 10000
D = 128
EPS = 1e-5

NC = 2
NS = 16
NW = NC * NS
C = 64
NCHUNK = 160
DEPTH = 4
EPT = C * NCHUNK
E_PAD = NW * EPT
AGG_ROWS = 10240
ZROWS_PER_TILE = AGG_ROWS // NS


def _matT(x, w):
    return lax.dot_general(x, w, (((1,), (1,)), ((), ())),
                           preferred_element_type=jnp.float32)


def _lin_body(x_ref, w_ref, b_ref, o_ref):
    o_ref[...] = _matT(x_ref[...], w_ref[...]) + b_ref[...][None, :]


def _tc_linear(x, w, b):
    nblk = 10
    rows = N_NODES // nblk
    return pl.pallas_call(
        _lin_body,
        grid=(nblk,),
        in_specs=[
            pl.BlockSpec((rows, D), lambda i: (i, 0)),
            pl.BlockSpec((D, D), lambda i: (0, 0)),
            pl.BlockSpec((D,), lambda i: (0,)),
        ],
        out_specs=pl.BlockSpec((rows, D), lambda i: (i, 0)),
        out_shape=jax.ShapeDtypeStruct((N_NODES, D), jnp.float32),
    )(x, w, b)


@functools.lru_cache(maxsize=1)
def _sc_scatter_build():
    mesh = plsc.VectorSubcoreMesh(core_axis_name="c", subcore_axis_name="s")

    scratch = (
        [pltpu.VMEM((NCHUNK // 4, C), jnp.int32)] * 2
        + [pltpu.VMEM((C, D), jnp.float32)] * DEPTH
        + [pltpu.VMEM_SHARED((AGG_ROWS, D), jnp.float32)]
        + [pltpu.SemaphoreType.DMA] * (2 * DEPTH + 2)
    )

    NST = 4
    SPT = NCHUNK // NST
    NIT = SPT // DEPTH

    @functools.partial(
        pl.kernel,
        mesh=mesh,
        out_type=jax.ShapeDtypeStruct((NC, AGG_ROWS, D), jnp.float32),
        scratch_types=scratch,
    )
    def sc_scatter(h_hbm, src_hbm, dst_hbm, zrows_hbm, out_hbm,
                   sidx, didx, *rest):
        bufs = rest[:DEPTH]
        agg = rest[DEPTH]
        gsem = rest[DEPTH + 1:DEPTH + 1 + DEPTH]
        ssem = rest[DEPTH + 1 + DEPTH:DEPTH + 1 + 2 * DEPTH]
        isem = rest[DEPTH + 1 + 2 * DEPTH:]
        cid = lax.axis_index("c")
        sid = lax.axis_index("s")
        wid = cid * NS + sid

        def icp_s(hh):
            return pltpu.make_async_copy(
                src_hbm.at[wid, pl.ds(hh * SPT, SPT)], sidx, isem[0])

        def icp_d(hh):
            return pltpu.make_async_copy(
                dst_hbm.at[wid, pl.ds(hh * SPT, SPT)], didx, isem[1])

        def gat(k, b):
            return pltpu.make_async_copy(h_hbm.at[sidx.at[k]], bufs[b], gsem[b])

        def sca(k, b):
            return pltpu.make_async_copy(bufs[b], agg.at[didx.at[k]], ssem[b])

        icp_s(0).start()
        icp_d(0).start()

        pltpu.sync_copy(zrows_hbm, bufs[0])
        for k in range(ZROWS_PER_TILE // C):
            pltpu.sync_copy(bufs[0],
                            agg.at[pl.ds(sid * ZROWS_PER_TILE + k * C, C)])
        icp_s(0).wait()
        for b in range(DEPTH):
            gat(b, b).start()
        icp_d(0).wait()
        plsc.subcore_barrier()

        for hh in range(NST):
            if hh > 0:
                icp_s(hh).wait()
                for b in range(DEPTH):
                    gat(b, b).start()
                icp_d(hh).wait()

            def body(j, carry, hh=hh):
                for b in range(DEPTH):
                    k = DEPTH * j + b
                    gat(k, b).wait()
                    sca(k, b).start(add=True)
                if hh < NST - 1:
                    @pl.when(j == NIT - 1)
                    def _prefetch_sidx():
                        icp_s(hh + 1).start()

                for b in range(DEPTH):
                    k = DEPTH * j + b
                    sca(k, b).wait()

                    @pl.when(j < NIT - 1)
                    def _prefetch():
                        gat(k + DEPTH, b).start()

                if hh < NST - 1:
                    @pl.when(j == NIT - 1)
                    def _prefetch_didx():
                        icp_d(hh + 1).start()

                return carry

            lax.fori_loop(0, NIT, body, 0)
        plsc.subcore_barrier()

        r0 = sid * ZROWS_PER_TILE
        pltpu.sync_copy(agg.at[pl.ds(r0, ZROWS_PER_TILE)],
                        out_hbm.at[cid, pl.ds(r0, ZROWS_PER_TILE)])

    return sc_scatter


_NBLK = 10
_ROWS = N_NODES // _NBLK


def _comb_body(p_ref, x_ref, w_ref, b_ref, g_ref, bt_ref, o_ref,
               t_ref, s_ref, q_ref):
    ph = pl.program_id(0)
    i = pl.program_id(1)

    @pl.when(ph == 0)
    def _compute():
        new = jnp.maximum(p_ref[0] + p_ref[1], 0.0)
        res = jnp.maximum(_matT(x_ref[...], w_ref[...]) + b_ref[...][None, :],
                          0.0)
        t = new + res
        t_ref[pl.ds(i * _ROWS, _ROWS), :] = t
        s_ref[pl.ds(i, 1), :] = jnp.sum(t, axis=0, keepdims=True)
        q_ref[pl.ds(i, 1), :] = jnp.sum(t * t, axis=0, keepdims=True)

    @pl.when(ph == 1)
    def _normalize():
        n = float(N_NODES)
        mean = jnp.sum(s_ref[...], axis=0, keepdims=True) / n
        var = jnp.sum(q_ref[...], axis=0, keepdims=True) / n - mean * mean
        inv = lax.rsqrt(var + EPS)
        t = t_ref[pl.ds(i * _ROWS, _ROWS), :]
        o_ref[...] = (t - mean) * (inv * g_ref[...][None, :]) + bt_ref[...][None, :]


def _tc_combine(p, x, w, b, gamma, beta):
    return pl.pallas_call(
        _comb_body,
        grid=(2, _NBLK),
        in_specs=[
            pl.BlockSpec((NC, _ROWS, D), lambda p_, i: (0, (1 - p_) * i, 0)),
            pl.BlockSpec((_ROWS, D), lambda p_, i: ((1 - p_) * i, 0)),
            pl.BlockSpec((D, D), lambda p_, i: (0, 0)),
            pl.BlockSpec((D,), lambda p_, i: (0,)),
            pl.BlockSpec((D,), lambda p_, i: (0,)),
            pl.BlockSpec((D,), lambda p_, i: (0,)),
        ],
        out_specs=pl.BlockSpec((_ROWS, D), lambda p_, i: (p_ * i, 0)),
        out_shape=jax.ShapeDtypeStruct((N_NODES, D), jnp.float32),
        scratch_shapes=[
            pltpu.VMEM((N_NODES, D), jnp.float32),
            pltpu.VMEM((_NBLK, D), jnp.float32),
            pltpu.VMEM((_NBLK, D), jnp.float32),
        ],
    )(p, x, w, b, gamma, beta)


def kernel(feats, edge_index, W_rel, b_rel, W_res, b_res, gamma, beta):
    src = edge_index[0].astype(jnp.int32)
    dst = edge_index[1].astype(jnp.int32)
    pad = E_PAD - src.shape[0]
    pad_iota = jnp.arange(pad, dtype=jnp.int32)
    src = jnp.concatenate([src, pad_iota % N_NODES])
    dst = jnp.concatenate([dst, N_NODES + pad_iota % (AGG_ROWS - N_NODES)])
    src3 = src.reshape(NW, NCHUNK, C)
    dst3 = dst.reshape(NW, NCHUNK, C)
    zrows = jnp.zeros((C, D), jnp.float32)

    h = _tc_linear(feats, W_rel, b_rel)
    p = _sc_scatter_build()(h, src3, dst3, zrows)
    return _tc_combine(p, feats, W_res, b_res, gamma, beta)
